# initial kernel scaffold (unmeasured)
import jax
import jax.numpy as jnp
from jax import lax
from jax.experimental import pallas as pl
from jax.experimental.pallas import tpu as pltpu

N_DEV = 16


def kernel(x, Wq, K_ext, V_ext, Wo):
    B, Sq, Dm = x.shape
    _, Skv_l, H, Dh = K_ext.shape
    Hl = Wq.shape[1] // Dh
    LHD = Hl * Dh
    Skv_g = Skv_l * N_DEV
    Dout = Wo.shape[1]
    R = B * Sq
    CH = R // N_DEV

    x2d = x.reshape(R, Dm)
    K2 = K_ext.reshape(B, Skv_l, H * Dh)
    V2 = V_ext.reshape(B, Skv_l, H * Dh)

    def body(x_ref, wq_ref, k_ref, v_ref, wo_ref, out_ref,
             kvpack, kvg, ctx2d, partial, red_ref, rs_buf,
             send_sems, kv_recv, rs_recv, ag_recv):
        me = lax.axis_index("i")

        for d in range(N_DEV):
            kvpack[d, 0:B] = k_ref[:, :, d * LHD:(d + 1) * LHD]
            kvpack[d, B:2 * B] = v_ref[:, :, d * LHD:(d + 1) * LHD]

        kvg[:, pl.ds(me * Skv_l, Skv_l), :] = kvpack[me]

        kv_rdmas = []
        for t in range(1, N_DEV):
            dest = (me + t) % N_DEV
            rd = pltpu.make_async_remote_copy(
                src_ref=kvpack.at[dest],
                dst_ref=kvg.at[:, pl.ds(me * Skv_l, Skv_l), :],
                send_sem=send_sems.at[t],
                recv_sem=kv_recv.at[t],
                device_id=(dest,),
                device_id_type=pl.DeviceIdType.MESH,
            )
            rd.start()
            kv_rdmas.append(rd)

        q2d = jnp.dot(x_ref[...], wq_ref[...],
                      preferred_element_type=jnp.float32)
        qb = lax.broadcasted_iota(jnp.int32, (Sq, Skv_g), 0) // 64
        kb = lax.broadcasted_iota(jnp.int32, (Sq, Skv_g), 1) // 64
        keep = (qb == kb) | (kb == 0) | ((qb + kb) % 3 == 0)
        bias = jnp.where(keep, 0.0, -1e9).astype(jnp.float32)

        for rd in kv_rdmas:
            rd.wait()

        for b in range(B):
            for h in range(Hl):
                q = q2d[b * Sq:(b + 1) * Sq, h * Dh:(h + 1) * Dh]
                k = kvg[b, :, h * Dh:(h + 1) * Dh]
                v = kvg[B + b, :, h * Dh:(h + 1) * Dh]
                s = lax.dot_general(q, k, (((1,), (1,)), ((), ())),
                                    preferred_element_type=jnp.float32)
                s = s * 0.125 + bias
                m = jnp.max(s, axis=1, keepdims=True)
                w = jnp.exp(s - m)
                w = w / jnp.sum(w, axis=1, keepdims=True)
                ctx = lax.dot_general(w, v, (((1,), (0,)), ((), ())),
                                      preferred_element_type=jnp.float32)
                ctx2d[b * Sq:(b + 1) * Sq, h * Dh:(h + 1) * Dh] = ctx

        partial[...] = jnp.dot(ctx2d[...], wo_ref[...],
                               preferred_element_type=jnp.float32)

        rs_rdmas = []
        for t in range(1, N_DEV):
            dest = (me + t) % N_DEV
            rd = pltpu.make_async_remote_copy(
                src_ref=partial.at[pl.ds(dest * CH, CH), :],
                dst_ref=rs_buf.at[t],
                send_sem=send_sems.at[t],
                recv_sem=rs_recv.at[t],
                device_id=(dest,),
                device_id_type=pl.DeviceIdType.MESH,
            )
            rd.start()
            rs_rdmas.append(rd)
        red = partial[pl.ds(me * CH, CH), :]
        for t in range(1, N_DEV):
            rs_rdmas[t - 1].wait()
            red = red + rs_buf[t]
        red_ref[...] = red
        out_ref[pl.ds(me * CH, CH), :] = red

        ag_rdmas = []
        for t in range(1, N_DEV):
            dest = (me + t) % N_DEV
            rd = pltpu.make_async_remote_copy(
                src_ref=red_ref,
                dst_ref=out_ref.at[pl.ds(me * CH, CH), :],
                send_sem=send_sems.at[t],
                recv_sem=ag_recv.at[t],
                device_id=(dest,),
                device_id_type=pl.DeviceIdType.MESH,
            )
            rd.start()
            ag_rdmas.append(rd)
        for rd in ag_rdmas:
            rd.wait()

    out2d = pl.pallas_call(
        body,
        out_shape=jax.ShapeDtypeStruct((R, Dout), jnp.float32),
        in_specs=[pl.BlockSpec(memory_space=pltpu.VMEM)] * 5,
        out_specs=pl.BlockSpec(memory_space=pltpu.VMEM),
        scratch_shapes=[
            pltpu.VMEM((N_DEV, 2 * B, Skv_l, LHD), jnp.float32),
            pltpu.VMEM((2 * B, Skv_g, LHD), jnp.float32),
            pltpu.VMEM((R, LHD), jnp.float32),
            pltpu.VMEM((R, Dout), jnp.float32),
            pltpu.VMEM((CH, Dout), jnp.float32),
            pltpu.VMEM((N_DEV, CH, Dout), jnp.float32),
            pltpu.SemaphoreType.DMA((N_DEV,)),
            pltpu.SemaphoreType.DMA((N_DEV,)),
            pltpu.SemaphoreType.DMA((N_DEV,)),
            pltpu.SemaphoreType.DMA((N_DEV,)),
        ],
        compiler_params=pltpu.CompilerParams(collective_id=0),
    )(x2d, Wq, K2, V2, Wo)

    return out2d.reshape(B, Sq, Dout)


# baseline (device time: 142540 ns/iter reference)
import jax
import jax.numpy as jnp
from jax import lax
from jax.experimental import pallas as pl
from jax.experimental.pallas import tpu as pltpu

N_DEV = 16


def kernel(x, Wq, K_ext, V_ext, Wo):
    B, Sq, Dm = x.shape
    _, Skv_l, H, Dh = K_ext.shape
    Hl = Wq.shape[1] // Dh
    LHD = Hl * Dh
    Skv_g = Skv_l * N_DEV
    Dout = Wo.shape[1]
    R = B * Sq
    CH = R // N_DEV

    x2d = x.reshape(R, Dm)
    K2 = K_ext.reshape(B, Skv_l, H * Dh)
    V2 = V_ext.reshape(B, Skv_l, H * Dh)

    def body(x_ref, wq_ref, k_ref, v_ref, wo_ref, out_ref,
             kvpack, kvg, ctx2d, partial, red_ref, rs_buf,
             send_sems, kv_recv, rs_recv, ag_recv):
        me = lax.axis_index("i")

        for d in range(N_DEV):
            kvpack[d, 0:B] = k_ref[:, :, d * LHD:(d + 1) * LHD]
            kvpack[d, B:2 * B] = v_ref[:, :, d * LHD:(d + 1) * LHD]

        kvg[:, pl.ds(me * Skv_l, Skv_l), :] = kvpack[me]

        kv_rdmas = []
        for t in range(1, N_DEV):
            dest = (me + t) % N_DEV
            rd = pltpu.make_async_remote_copy(
                src_ref=kvpack.at[dest],
                dst_ref=kvg.at[:, pl.ds(me * Skv_l, Skv_l), :],
                send_sem=send_sems.at[t],
                recv_sem=kv_recv.at[t],
                device_id=(dest,),
                device_id_type=pl.DeviceIdType.MESH,
            )
            rd.start()
            kv_rdmas.append(rd)

        q2d = jnp.dot(x_ref[...], wq_ref[...],
                      preferred_element_type=jnp.float32)
        qb = lax.broadcasted_iota(jnp.int32, (Sq, Skv_g), 0) // 64
        kb = lax.broadcasted_iota(jnp.int32, (Sq, Skv_g), 1) // 64
        keep = (qb == kb) | (kb == 0) | ((qb + kb) % 3 == 0)
        bias = jnp.where(keep, 0.0, -1e9).astype(jnp.float32)

        for rd in kv_rdmas:
            rd.wait()

        for b in range(B):
            for h in range(Hl):
                q = q2d[b * Sq:(b + 1) * Sq, h * Dh:(h + 1) * Dh]
                k = kvg[b, :, h * Dh:(h + 1) * Dh]
                v = kvg[B + b, :, h * Dh:(h + 1) * Dh]
                s = lax.dot_general(q, k, (((1,), (1,)), ((), ())),
                                    preferred_element_type=jnp.float32)
                s = s * 0.125 + bias
                m = jnp.max(s, axis=1, keepdims=True)
                w = jnp.exp(s - m)
                w = w / jnp.sum(w, axis=1, keepdims=True)
                ctx = lax.dot_general(w, v, (((1,), (0,)), ((), ())),
                                      preferred_element_type=jnp.float32)
                ctx2d[b * Sq:(b + 1) * Sq, h * Dh:(h + 1) * Dh] = ctx

        partial[...] = jnp.dot(ctx2d[...], wo_ref[...],
                               preferred_element_type=jnp.float32)

        rs_rdmas = []
        for t in range(1, N_DEV):
            dest = (me + t) % N_DEV
            rd = pltpu.make_async_remote_copy(
                src_ref=partial.at[pl.ds(dest * CH, CH), :],
                dst_ref=rs_buf.at[t],
                send_sem=send_sems.at[t],
                recv_sem=rs_recv.at[t],
                device_id=(dest,),
                device_id_type=pl.DeviceIdType.MESH,
            )
            rd.start()
            rs_rdmas.append(rd)
        red = partial[pl.ds(me * CH, CH), :]
        for t in range(1, N_DEV):
            rs_rdmas[t - 1].wait()
            red = red + rs_buf[t]
        red_ref[...] = red
        out_ref[pl.ds(me * CH, CH), :] = red

        ag_rdmas = []
        for t in range(1, N_DEV):
            dest = (me + t) % N_DEV
            rd = pltpu.make_async_remote_copy(
                src_ref=red_ref,
                dst_ref=out_ref.at[pl.ds(me * CH, CH), :],
                send_sem=send_sems.at[t],
                recv_sem=ag_recv.at[t],
                device_id=(dest,),
                device_id_type=pl.DeviceIdType.MESH,
            )
            rd.start()
            ag_rdmas.append(rd)
        for rd in ag_rdmas:
            rd.wait()

    out2d = pl.pallas_call(
        body,
        out_shape=jax.ShapeDtypeStruct((R, Dout), jnp.float32),
        in_specs=[pl.BlockSpec(memory_space=pltpu.VMEM)] * 5,
        out_specs=pl.BlockSpec(memory_space=pltpu.VMEM),
        scratch_shapes=[
            pltpu.VMEM((N_DEV, 2 * B, Skv_l, LHD), jnp.float32),
            pltpu.VMEM((2 * B, Skv_g, LHD), jnp.float32),
            pltpu.VMEM((R, LHD), jnp.float32),
            pltpu.VMEM((R, Dout), jnp.float32),
            pltpu.VMEM((CH, Dout), jnp.float32),
            pltpu.VMEM((N_DEV, CH, Dout), jnp.float32),
            pltpu.SemaphoreType.DMA((N_DEV,)),
            pltpu.SemaphoreType.DMA((N_DEV,)),
            pltpu.SemaphoreType.DMA((N_DEV,)),
            pltpu.SemaphoreType.DMA((N_DEV,)),
        ],
    )(x2d, Wq, K2, V2, Wo)

    return out2d.reshape(B, Sq, Dout)


# device time: 88660 ns/iter; 1.6077x vs baseline; 1.6077x over previous
import jax
import jax.numpy as jnp
from jax import lax
from jax.experimental import pallas as pl
from jax.experimental.pallas import tpu as pltpu

N_DEV = 16


def kernel(x, Wq, K_ext, V_ext, Wo):
    B, Sq, Dm = x.shape
    _, Skv_l, H, Dh = K_ext.shape
    Hl = Wq.shape[1] // Dh
    LHD = Hl * Dh
    Skv_g = Skv_l * N_DEV
    Dout = Wo.shape[1]
    R = B * Sq
    CH = R // N_DEV

    x2d = x.reshape(R, Dm)
    K2 = K_ext.reshape(B, Skv_l, H * Dh)
    V2 = V_ext.reshape(B, Skv_l, H * Dh)

    def body(x_ref, wq_ref, k_ref, v_ref, wo_ref, out_ref,
             kvpack, kvg, ctx2d, partial, red_ref, rs_buf,
             send_sems, kv_recv, rs_recv, ag_recv):
        me = lax.axis_index("i")

        for d in range(N_DEV):
            kvpack[d, 0:B] = k_ref[:, :, d * LHD:(d + 1) * LHD].astype(
                jnp.bfloat16)
            kvpack[d, B:2 * B] = v_ref[:, :, d * LHD:(d + 1) * LHD].astype(
                jnp.bfloat16)

        kvg[:, pl.ds(me * Skv_l, Skv_l), :] = kvpack[me]

        kv_rdmas = []
        for t in range(1, N_DEV):
            dest = (me + t) % N_DEV
            rd = pltpu.make_async_remote_copy(
                src_ref=kvpack.at[dest],
                dst_ref=kvg.at[:, pl.ds(me * Skv_l, Skv_l), :],
                send_sem=send_sems.at[t],
                recv_sem=kv_recv.at[t],
                device_id=(dest,),
                device_id_type=pl.DeviceIdType.MESH,
            )
            rd.start()
            kv_rdmas.append(rd)

        q2d = jnp.dot(x_ref[...], wq_ref[...],
                      preferred_element_type=jnp.float32)
        qb = lax.broadcasted_iota(jnp.int32, (Sq, Skv_g), 0) // 64
        kb = lax.broadcasted_iota(jnp.int32, (Sq, Skv_g), 1) // 64
        keep = (qb == kb) | (kb == 0) | ((qb + kb) % 3 == 0)
        bias = jnp.where(keep, 0.0, -1e9).astype(jnp.float32)

        for rd in kv_rdmas:
            rd.wait()

        for b in range(B):
            for h in range(Hl):
                q = q2d[b * Sq:(b + 1) * Sq, h * Dh:(h + 1) * Dh].astype(
                    jnp.bfloat16)
                k = kvg[b, :, h * Dh:(h + 1) * Dh]
                v = kvg[B + b, :, h * Dh:(h + 1) * Dh]
                s = lax.dot_general(q, k, (((1,), (1,)), ((), ())),
                                    preferred_element_type=jnp.float32)
                s = s * 0.125 + bias
                m = jnp.max(s, axis=1, keepdims=True)
                w = jnp.exp(s - m)
                w = (w / jnp.sum(w, axis=1, keepdims=True)).astype(jnp.bfloat16)
                ctx = lax.dot_general(w, v, (((1,), (0,)), ((), ())),
                                      preferred_element_type=jnp.float32)
                ctx2d[b * Sq:(b + 1) * Sq, h * Dh:(h + 1) * Dh] = ctx

        partial[...] = jnp.dot(ctx2d[...], wo_ref[...],
                               preferred_element_type=jnp.float32)

        rs_rdmas = []
        for t in range(1, N_DEV):
            dest = (me + t) % N_DEV
            rd = pltpu.make_async_remote_copy(
                src_ref=partial.at[pl.ds(dest * CH, CH), :],
                dst_ref=rs_buf.at[t],
                send_sem=send_sems.at[t],
                recv_sem=rs_recv.at[t],
                device_id=(dest,),
                device_id_type=pl.DeviceIdType.MESH,
            )
            rd.start()
            rs_rdmas.append(rd)
        red = partial[pl.ds(me * CH, CH), :]
        for t in range(1, N_DEV):
            rs_rdmas[t - 1].wait()
            red = red + rs_buf[t]
        red_ref[...] = red
        out_ref[pl.ds(me * CH, CH), :] = red

        ag_rdmas = []
        for t in range(1, N_DEV):
            dest = (me + t) % N_DEV
            rd = pltpu.make_async_remote_copy(
                src_ref=red_ref,
                dst_ref=out_ref.at[pl.ds(me * CH, CH), :],
                send_sem=send_sems.at[t],
                recv_sem=ag_recv.at[t],
                device_id=(dest,),
                device_id_type=pl.DeviceIdType.MESH,
            )
            rd.start()
            ag_rdmas.append(rd)
        for rd in ag_rdmas:
            rd.wait()

    out2d = pl.pallas_call(
        body,
        out_shape=jax.ShapeDtypeStruct((R, Dout), jnp.float32),
        in_specs=[pl.BlockSpec(memory_space=pltpu.VMEM)] * 5,
        out_specs=pl.BlockSpec(memory_space=pltpu.VMEM),
        scratch_shapes=[
            pltpu.VMEM((N_DEV, 2 * B, Skv_l, LHD), jnp.bfloat16),
            pltpu.VMEM((2 * B, Skv_g, LHD), jnp.bfloat16),
            pltpu.VMEM((R, LHD), jnp.float32),
            pltpu.VMEM((R, Dout), jnp.float32),
            pltpu.VMEM((CH, Dout), jnp.float32),
            pltpu.VMEM((N_DEV, CH, Dout), jnp.float32),
            pltpu.SemaphoreType.DMA((N_DEV,)),
            pltpu.SemaphoreType.DMA((N_DEV,)),
            pltpu.SemaphoreType.DMA((N_DEV,)),
            pltpu.SemaphoreType.DMA((N_DEV,)),
        ],
    )(x2d, Wq, K2, V2, Wo)

    return out2d.reshape(B, Sq, Dout)


# device time: 80611 ns/iter; 1.7682x vs baseline; 1.0998x over previous
import jax
import jax.numpy as jnp
from jax import lax
from jax.experimental import pallas as pl
from jax.experimental.pallas import tpu as pltpu

N_DEV = 16
BLK = 64


def kernel(x, Wq, K_ext, V_ext, Wo):
    B, Sq, Dm = x.shape
    _, Skv_l, H, Dh = K_ext.shape
    Hl = Wq.shape[1] // Dh
    LHD = Hl * Dh
    Skv_g = Skv_l * N_DEV
    Dout = Wo.shape[1]
    R = B * Sq
    CH = R // N_DEV
    NB = Skv_l // BLK

    x2d = x.reshape(R, Dm)
    K2 = K_ext.reshape(B, Skv_l, H * Dh)
    V2 = V_ext.reshape(B, Skv_l, H * Dh)

    def blk_needed(j, blk):
        kb = 2 * j + blk
        return (kb < 2) | (kb % 3 != 1)

    dead_blocks = [g for g in range(Skv_g // BLK)
                   if g >= 2 and g % 3 == 1]

    def body(x_ref, wq_ref, k_ref, v_ref, wo_ref, out_ref,
             kvpack, kvg, ctx2d, partial_bf, red_ref, out_bf, rs_buf,
             kv_send, kv_recv, ar_send, rs_recv, ag_recv):
        me = lax.axis_index("i")

        for d in range(N_DEV):
            kvpack[d, 0:B] = k_ref[:, :, d * LHD:(d + 1) * LHD].astype(
                jnp.bfloat16)
            kvpack[d, B:2 * B] = v_ref[:, :, d * LHD:(d + 1) * LHD].astype(
                jnp.bfloat16)

        kvg[:, pl.ds(me * Skv_l, Skv_l), :] = kvpack[me]

        for g in dead_blocks:
            kvg[B:2 * B, g * BLK:(g + 1) * BLK, :] = jnp.zeros(
                (B, BLK, LHD), jnp.bfloat16)

        send_descs = []
        for t in range(1, N_DEV):
            dest = (me + t) % N_DEV
            src = (me - t) % N_DEV
            for blk in range(NB):
                sd = pltpu.make_async_remote_copy(
                    src_ref=kvpack.at[dest, :, blk * BLK:(blk + 1) * BLK, :],
                    dst_ref=kvg.at[:, pl.ds(me * Skv_l + blk * BLK, BLK), :],
                    send_sem=kv_send.at[t, blk],
                    recv_sem=kv_recv.at[t, blk],
                    device_id=(dest,),
                    device_id_type=pl.DeviceIdType.MESH,
                )
                pl.when(blk_needed(me, blk))(sd.start)
                send_descs.append(sd)

        q2d = jnp.dot(x_ref[...], wq_ref[...],
                      preferred_element_type=jnp.float32)
        qb = lax.broadcasted_iota(jnp.int32, (Sq, Skv_g), 0) // BLK
        kb = lax.broadcasted_iota(jnp.int32, (Sq, Skv_g), 1) // BLK
        keep = (qb == kb) | (kb == 0) | ((qb + kb) % 3 == 0)

        for t in range(1, N_DEV):
            dest = (me + t) % N_DEV
            src = (me - t) % N_DEV
            for blk in range(NB):
                rv = pltpu.make_async_remote_copy(
                    src_ref=kvpack.at[dest, :, blk * BLK:(blk + 1) * BLK, :],
                    dst_ref=kvg.at[:, pl.ds(src * Skv_l + blk * BLK, BLK), :],
                    send_sem=kv_send.at[t, blk],
                    recv_sem=kv_recv.at[t, blk],
                    device_id=(dest,),
                    device_id_type=pl.DeviceIdType.MESH,
                )
                pl.when(blk_needed(src, blk))(rv.wait_recv)

        for b in range(B):
            for h in range(Hl):
                q = q2d[b * Sq:(b + 1) * Sq, h * Dh:(h + 1) * Dh].astype(
                    jnp.bfloat16)
                k = kvg[b, :, h * Dh:(h + 1) * Dh]
                v = kvg[B + b, :, h * Dh:(h + 1) * Dh]
                s = lax.dot_general(q, k, (((1,), (1,)), ((), ())),
                                    preferred_element_type=jnp.float32)
                s = jnp.where(keep, s * 0.125, -1e9)
                m = jnp.max(s, axis=1, keepdims=True)
                w = jnp.exp(s - m)
                w = (w / jnp.sum(w, axis=1, keepdims=True)).astype(jnp.bfloat16)
                ctx = lax.dot_general(w, v, (((1,), (0,)), ((), ())),
                                      preferred_element_type=jnp.float32)
                ctx2d[b * Sq:(b + 1) * Sq, h * Dh:(h + 1) * Dh] = ctx

        partial_bf[...] = jnp.dot(ctx2d[...], wo_ref[...],
                                  preferred_element_type=jnp.float32
                                  ).astype(jnp.bfloat16)

        rs_rdmas = []
        for t in range(1, N_DEV):
            dest = (me + t) % N_DEV
            rd = pltpu.make_async_remote_copy(
                src_ref=partial_bf.at[pl.ds(dest * CH, CH), :],
                dst_ref=rs_buf.at[t],
                send_sem=ar_send.at[t],
                recv_sem=rs_recv.at[t],
                device_id=(dest,),
                device_id_type=pl.DeviceIdType.MESH,
            )
            rd.start()
            rs_rdmas.append(rd)
        red = partial_bf[pl.ds(me * CH, CH), :].astype(jnp.float32)
        for t in range(1, N_DEV):
            rs_rdmas[t - 1].wait()
            red = red + rs_buf[t].astype(jnp.float32)
        red_ref[...] = red.astype(jnp.bfloat16)
        out_bf[pl.ds(me * CH, CH), :] = red_ref[...]

        ag_rdmas = []
        for t in range(1, N_DEV):
            dest = (me + t) % N_DEV
            rd = pltpu.make_async_remote_copy(
                src_ref=red_ref,
                dst_ref=out_bf.at[pl.ds(me * CH, CH), :],
                send_sem=ar_send.at[t],
                recv_sem=ag_recv.at[t],
                device_id=(dest,),
                device_id_type=pl.DeviceIdType.MESH,
            )
            rd.start()
            ag_rdmas.append(rd)
        for rd in ag_rdmas:
            rd.wait()
        out_ref[...] = out_bf[...].astype(jnp.float32)

        for i, sd in enumerate(send_descs):
            blk = i % NB
            pl.when(blk_needed(me, blk))(sd.wait_send)

    out2d = pl.pallas_call(
        body,
        out_shape=jax.ShapeDtypeStruct((R, Dout), jnp.float32),
        in_specs=[pl.BlockSpec(memory_space=pltpu.VMEM)] * 5,
        out_specs=pl.BlockSpec(memory_space=pltpu.VMEM),
        scratch_shapes=[
            pltpu.VMEM((N_DEV, 2 * B, Skv_l, LHD), jnp.bfloat16),
            pltpu.VMEM((2 * B, Skv_g, LHD), jnp.bfloat16),
            pltpu.VMEM((R, LHD), jnp.float32),
            pltpu.VMEM((R, Dout), jnp.bfloat16),
            pltpu.VMEM((CH, Dout), jnp.bfloat16),
            pltpu.VMEM((R, Dout), jnp.bfloat16),
            pltpu.VMEM((N_DEV, CH, Dout), jnp.bfloat16),
            pltpu.SemaphoreType.DMA((N_DEV, 2)),
            pltpu.SemaphoreType.DMA((N_DEV, 2)),
            pltpu.SemaphoreType.DMA((N_DEV,)),
            pltpu.SemaphoreType.DMA((N_DEV,)),
            pltpu.SemaphoreType.DMA((N_DEV,)),
        ],
    )(x2d, Wq, K2, V2, Wo)

    return out2d.reshape(B, Sq, Dout)


# device time: 77849 ns/iter; 1.8310x vs baseline; 1.0355x over previous
import jax
import jax.numpy as jnp
from jax import lax
from jax.experimental import pallas as pl
from jax.experimental.pallas import tpu as pltpu

N_DEV = 16
BLK = 64
SPLIT = 8


def kernel(x, Wq, K_ext, V_ext, Wo):
    B, Sq, Dm = x.shape
    _, Skv_l, H, Dh = K_ext.shape
    Hl = Wq.shape[1] // Dh
    LHD = Hl * Dh
    Skv_g = Skv_l * N_DEV
    Dout = Wo.shape[1]
    R = B * Sq
    CH = R // N_DEV
    NB = Skv_l // BLK
    HALF = SPLIT * Skv_l

    x2d = x.reshape(R, Dm)
    K2 = K_ext.reshape(B, Skv_l, H * Dh)
    V2 = V_ext.reshape(B, Skv_l, H * Dh)

    def blk_needed(j, blk):
        kb = 2 * j + blk
        return (kb < 2) | (kb % 3 != 1)

    def body(x_ref, wq_ref, k_ref, v_ref, wo_ref, out_ref,
             kvpack, kvg, ctx2d, partial_bf, red_ref, out_bf, rs_buf,
             kv_send, kv_recv, ar_send, rs_recv, ag_recv):
        me = lax.axis_index("i")

        for d in range(N_DEV):
            kvpack[d, 0:B] = k_ref[:, :, d * LHD:(d + 1) * LHD].astype(
                jnp.bfloat16)
            kvpack[d, B:2 * B] = v_ref[:, :, d * LHD:(d + 1) * LHD].astype(
                jnp.bfloat16)

        kvg[:, 0:Skv_l, :] = kvpack[me]

        send_descs = []
        for t in range(1, N_DEV):
            dest = (me + t) % N_DEV
            for blk in range(NB):
                sd = pltpu.make_async_remote_copy(
                    src_ref=kvpack.at[dest, :, blk * BLK:(blk + 1) * BLK, :],
                    dst_ref=kvg.at[:, t * Skv_l + blk * BLK:
                                   t * Skv_l + (blk + 1) * BLK, :],
                    send_sem=kv_send.at[t, blk],
                    recv_sem=kv_recv.at[t, blk],
                    device_id=(dest,),
                    device_id_type=pl.DeviceIdType.MESH,
                )
                pl.when(blk_needed(me, blk))(sd.start)
                send_descs.append(sd)

        q2d = jnp.dot(x_ref[...], wq_ref[...],
                      preferred_element_type=jnp.float32)
        col = lax.broadcasted_iota(jnp.int32, (1, Skv_g), 1)
        chunk = (me - col // Skv_l) % N_DEV
        kb = 2 * chunk + (col % Skv_l) // BLK
        qb = lax.broadcasted_iota(jnp.int32, (Sq, 1), 0) // BLK
        keep = (qb == kb) | (kb == 0) | ((qb + kb) % 3 == 0)
        vmask = jnp.where((kb < 2) | (kb % 3 != 1), 1.0, 0.0).astype(
            jnp.bfloat16).reshape(Skv_g, 1)

        def recv_wait(t, blk):
            dest = (me + t) % N_DEV
            src = (me - t) % N_DEV
            rv = pltpu.make_async_remote_copy(
                src_ref=kvpack.at[dest, :, blk * BLK:(blk + 1) * BLK, :],
                dst_ref=kvg.at[:, t * Skv_l + blk * BLK:
                               t * Skv_l + (blk + 1) * BLK, :],
                send_sem=kv_send.at[t, blk],
                recv_sem=kv_recv.at[t, blk],
                device_id=(dest,),
                device_id_type=pl.DeviceIdType.MESH,
            )
            pl.when(blk_needed(src, blk))(rv.wait_recv)

        def stage(b, h, q, lo):
            k = kvg[b, lo:lo + HALF, h * Dh:(h + 1) * Dh]
            v = kvg[B + b, lo:lo + HALF, h * Dh:(h + 1) * Dh]
            v = v * vmask[lo:lo + HALF]
            s = lax.dot_general(q, k, (((1,), (1,)), ((), ())),
                                preferred_element_type=jnp.float32)
            s = jnp.where(keep[:, lo:lo + HALF], s * 0.125, -1e9)
            m = jnp.max(s, axis=1, keepdims=True)
            w = jnp.exp(s - m)
            l = jnp.sum(w, axis=1, keepdims=True)
            acc = lax.dot_general(w.astype(jnp.bfloat16), v,
                                  (((1,), (0,)), ((), ())),
                                  preferred_element_type=jnp.float32)
            return m, l, acc

        for t in range(1, SPLIT):
            for blk in range(NB):
                recv_wait(t, blk)
        statsA = []
        for b in range(B):
            for h in range(Hl):
                q = q2d[b * Sq:(b + 1) * Sq, h * Dh:(h + 1) * Dh].astype(
                    jnp.bfloat16)
                statsA.append((q, stage(b, h, q, 0)))

        for t in range(SPLIT, N_DEV):
            for blk in range(NB):
                recv_wait(t, blk)
        i = 0
        for b in range(B):
            for h in range(Hl):
                q, (mA, lA, accA) = statsA[i]
                i += 1
                mB, lB, accB = stage(b, h, q, HALF)
                m = jnp.maximum(mA, mB)
                fA = jnp.exp(mA - m)
                fB = jnp.exp(mB - m)
                ctx = (fA * accA + fB * accB) / (fA * lA + fB * lB)
                ctx2d[b * Sq:(b + 1) * Sq, h * Dh:(h + 1) * Dh] = ctx

        partial_bf[...] = jnp.dot(ctx2d[...], wo_ref[...],
                                  preferred_element_type=jnp.float32
                                  ).astype(jnp.bfloat16)

        rs_rdmas = []
        for t in range(1, N_DEV):
            dest = (me + t) % N_DEV
            rd = pltpu.make_async_remote_copy(
                src_ref=partial_bf.at[pl.ds(dest * CH, CH), :],
                dst_ref=rs_buf.at[t],
                send_sem=ar_send.at[t],
                recv_sem=rs_recv.at[t],
                device_id=(dest,),
                device_id_type=pl.DeviceIdType.MESH,
            )
            rd.start()
            rs_rdmas.append(rd)
        red = partial_bf[pl.ds(me * CH, CH), :].astype(jnp.float32)
        for t in range(1, N_DEV):
            rs_rdmas[t - 1].wait()
            red = red + rs_buf[t].astype(jnp.float32)
        red_ref[...] = red.astype(jnp.bfloat16)
        out_bf[pl.ds(me * CH, CH), :] = red_ref[...]

        ag_rdmas = []
        for t in range(1, N_DEV):
            dest = (me + t) % N_DEV
            rd = pltpu.make_async_remote_copy(
                src_ref=red_ref,
                dst_ref=out_bf.at[pl.ds(me * CH, CH), :],
                send_sem=ar_send.at[t],
                recv_sem=ag_recv.at[t],
                device_id=(dest,),
                device_id_type=pl.DeviceIdType.MESH,
            )
            rd.start()
            ag_rdmas.append(rd)
        for rd in ag_rdmas:
            rd.wait()
        out_ref[...] = out_bf[...].astype(jnp.float32)

        for i, sd in enumerate(send_descs):
            blk = i % NB
            pl.when(blk_needed(me, blk))(sd.wait_send)

    out2d = pl.pallas_call(
        body,
        out_shape=jax.ShapeDtypeStruct((R, Dout), jnp.float32),
        in_specs=[pl.BlockSpec(memory_space=pltpu.VMEM)] * 5,
        out_specs=pl.BlockSpec(memory_space=pltpu.VMEM),
        scratch_shapes=[
            pltpu.VMEM((N_DEV, 2 * B, Skv_l, LHD), jnp.bfloat16),
            pltpu.VMEM((2 * B, Skv_g, LHD), jnp.bfloat16),
            pltpu.VMEM((R, LHD), jnp.float32),
            pltpu.VMEM((R, Dout), jnp.bfloat16),
            pltpu.VMEM((CH, Dout), jnp.bfloat16),
            pltpu.VMEM((R, Dout), jnp.bfloat16),
            pltpu.VMEM((N_DEV, CH, Dout), jnp.bfloat16),
            pltpu.SemaphoreType.DMA((N_DEV, 2)),
            pltpu.SemaphoreType.DMA((N_DEV, 2)),
            pltpu.SemaphoreType.DMA((N_DEV,)),
            pltpu.SemaphoreType.DMA((N_DEV,)),
            pltpu.SemaphoreType.DMA((N_DEV,)),
        ],
    )(x2d, Wq, K2, V2, Wo)

    return out2d.reshape(B, Sq, Dout)
